# Initial kernel scaffold; baseline (speedup 1.0000x reference)
import jax
import jax.numpy as jnp
from jax.experimental import pallas as pl


def kernel(x, nbrs, W):
    raise NotImplementedError("write your pallas kernel here")


# speedup vs baseline: 9.0404x; 9.0404x over previous
import functools

import jax
import jax.numpy as jnp
from jax import lax
from jax.experimental import pallas as pl
from jax.experimental.pallas import tpu as pltpu
from jax.experimental.pallas import tpu_sc as plsc

_N, _DEG, _D = 10000, 32, 128
_NC, _NS, _L = 2, 16, 16
_NW = _NC * _NS
_PER_W = 320
_NPAD = _NW * _PER_W


def _oems_pairs(n):
    pairs = []

    def merge(lo, m, r):
        step = r * 2
        if step < m:
            merge(lo, m, step)
            merge(lo + r, m, step)
            for i in range(lo + r, lo + m - r, step):
                pairs.append((i, i + r))
        else:
            pairs.append((lo, lo + r))

    def sort(lo, m):
        if m > 1:
            half = m // 2
            sort(lo, half)
            sort(lo + half, half)
            merge(lo, m, 1)

    sort(0, n)
    return pairs


_PAIRS16 = _oems_pairs(16)


def _median32(vals):
    a = list(vals[:16])
    b = list(vals[16:])
    for (i, j) in _PAIRS16:
        alo = jnp.minimum(a[i], a[j])
        ahi = jnp.maximum(a[i], a[j])
        a[i], a[j] = alo, ahi
        blo = jnp.minimum(b[i], b[j])
        bhi = jnp.maximum(b[i], b[j])
        b[i], b[j] = blo, bhi
    m = jnp.minimum(b[15], a[15])
    for i in range(1, 16):
        m = jnp.minimum(m, jnp.maximum(a[i - 1], b[15 - i]))
    return m


def _mm_body(x_ref, w_ref, o_ref):
    o_ref[...] = lax.dot_general(
        x_ref[...], w_ref[...], (((1,), (1,)), ((), ())),
        preferred_element_type=jnp.float32)


_MM_BLK = 2000

_matmul = pl.pallas_call(
    _mm_body,
    out_shape=jax.ShapeDtypeStruct((_N, _D), jnp.float32),
    grid=(_N // _MM_BLK,),
    in_specs=[
        pl.BlockSpec((_MM_BLK, _D), lambda i: (i, 0)),
        pl.BlockSpec((_D, _D), lambda i: (0, 0)),
    ],
    out_specs=pl.BlockSpec((_MM_BLK, _D), lambda i: (i, 0)),
)

_mesh = plsc.VectorSubcoreMesh(core_axis_name="c", subcore_axis_name="s")


@functools.partial(
    pl.kernel,
    mesh=_mesh,
    out_type=jax.ShapeDtypeStruct((_NPAD, _D), jnp.float32),
    scratch_types=[
        pltpu.VMEM((_PER_W * _DEG,), jnp.int32),
        pltpu.VMEM((_DEG, _D), jnp.float32),
        pltpu.VMEM((1, _D), jnp.float32),
        pltpu.SemaphoreType.DMA,
    ],
)
def _sc_median(h_hbm, nbrs_hbm, out_hbm, idx_v, gbuf, orow, sem):
    wid = lax.axis_index("s") * _NC + lax.axis_index("c")
    nbase = wid * _PER_W
    pltpu.sync_copy(nbrs_hbm.at[pl.ds(nbase * _DEG, _PER_W * _DEG)], idx_v)

    def body(i, carry):
        pltpu.async_copy(
            h_hbm.at[idx_v.at[pl.ds(i * _DEG, _DEG)]], gbuf, sem).wait()
        for f in range(_D // _L):
            sl = pl.ds(f * _L, _L)
            vals = [gbuf[j, sl] for j in range(_DEG)]
            orow[0, sl] = _median32(vals)
        pltpu.sync_copy(orow, out_hbm.at[pl.ds(nbase + i, 1)])
        return carry

    lax.fori_loop(0, _PER_W, body, 0)


@jax.jit
def _run(x, nbrs, W):
    h = _matmul(x, W)
    nbrs32 = nbrs.astype(jnp.int32)
    nbrs_pad = jnp.zeros((_NPAD, _DEG), jnp.int32).at[:_N].set(nbrs32)
    out = _sc_median(h, nbrs_pad.reshape(-1))
    return out[:_N]


def kernel(x, nbrs, W):
    return _run(x, nbrs, W)


# speedup vs baseline: 13.6524x; 1.5102x over previous
import functools

import jax
import jax.numpy as jnp
from jax import lax
from jax.experimental import pallas as pl
from jax.experimental.pallas import tpu as pltpu
from jax.experimental.pallas import tpu_sc as plsc

_N, _DEG, _D = 10000, 32, 128
_NC, _NS, _L = 2, 16, 16
_NW = _NC * _NS
_PER_W = 320
_NPAD = _NW * _PER_W


def _oems_pairs(n):
    pairs = []

    def merge(lo, m, r):
        step = r * 2
        if step < m:
            merge(lo, m, step)
            merge(lo + r, m, step)
            for i in range(lo + r, lo + m - r, step):
                pairs.append((i, i + r))
        else:
            pairs.append((lo, lo + r))

    def sort(lo, m):
        if m > 1:
            half = m // 2
            sort(lo, half)
            sort(lo + half, half)
            merge(lo, m, 1)

    sort(0, n)
    return pairs


_PAIRS16 = _oems_pairs(16)


def _median32(vals):
    a = list(vals[:16])
    b = list(vals[16:])
    for (i, j) in _PAIRS16:
        alo = jnp.minimum(a[i], a[j])
        ahi = jnp.maximum(a[i], a[j])
        a[i], a[j] = alo, ahi
        blo = jnp.minimum(b[i], b[j])
        bhi = jnp.maximum(b[i], b[j])
        b[i], b[j] = blo, bhi
    m = jnp.minimum(b[15], a[15])
    for i in range(1, 16):
        m = jnp.minimum(m, jnp.maximum(a[i - 1], b[15 - i]))
    return m


def _mm_body(x_ref, w_ref, o_ref):
    o_ref[...] = lax.dot_general(
        x_ref[...], w_ref[...], (((1,), (1,)), ((), ())),
        preferred_element_type=jnp.float32)


_MM_BLK = 2000

_matmul = pl.pallas_call(
    _mm_body,
    out_shape=jax.ShapeDtypeStruct((_N, _D), jnp.float32),
    grid=(_N // _MM_BLK,),
    in_specs=[
        pl.BlockSpec((_MM_BLK, _D), lambda i: (i, 0)),
        pl.BlockSpec((_D, _D), lambda i: (0, 0)),
    ],
    out_specs=pl.BlockSpec((_MM_BLK, _D), lambda i: (i, 0)),
)

_mesh = plsc.VectorSubcoreMesh(core_axis_name="c", subcore_axis_name="s")


_RING = 4


@functools.partial(
    pl.kernel,
    mesh=_mesh,
    out_type=jax.ShapeDtypeStruct((_NPAD, _D), jnp.float32),
    scratch_types=[
        pltpu.VMEM((_PER_W * _DEG,), jnp.int32),
        pltpu.VMEM((_RING, _DEG, _D), jnp.float32),
        pltpu.VMEM((_RING, 1, _D), jnp.float32),
    ]
    + [pltpu.SemaphoreType.DMA] * (2 * _RING),
)
def _sc_median(h_hbm, nbrs_hbm, out_hbm, idx_v, gbuf, obuf, *sems):
    gsems = sems[:_RING]
    wsems = sems[_RING:]
    wid = lax.axis_index("s") * _NC + lax.axis_index("c")
    nbase = wid * _PER_W
    pltpu.sync_copy(nbrs_hbm.at[pl.ds(nbase * _DEG, _PER_W * _DEG)], idx_v)

    def gcopy(i, b, sem):
        return pltpu.make_async_copy(
            h_hbm.at[idx_v.at[pl.ds(i * _DEG, _DEG)]], gbuf.at[b], sem)

    for b in range(_RING - 1):
        gcopy(b, b, gsems[b]).start()

    def body(g, carry):
        for b in range(_RING):
            i = g * _RING + b
            gcopy(i, b, gsems[b]).wait()
            nb = (b + _RING - 1) % _RING

            @pl.when(i + _RING - 1 < _PER_W)
            def _():
                gcopy(i + _RING - 1, nb, gsems[nb]).start()

            @pl.when(g > 0)
            def _():
                pltpu.make_async_copy(
                    obuf.at[b], out_hbm.at[pl.ds(nbase + i - _RING, 1)],
                    wsems[b]).wait()

            for f in range(_D // _L):
                sl = pl.ds(f * _L, _L)
                obuf[b, 0, sl] = _median32([gbuf[b, j, sl] for j in range(_DEG)])
            pltpu.make_async_copy(
                obuf.at[b], out_hbm.at[pl.ds(nbase + i, 1)], wsems[b]).start()
        return carry

    lax.fori_loop(0, _PER_W // _RING, body, 0)
    for b in range(_RING):
        pltpu.make_async_copy(
            obuf.at[b], out_hbm.at[pl.ds(nbase + _PER_W - _RING + b, 1)],
            wsems[b]).wait()


@jax.jit
def _run(x, nbrs, W):
    h = _matmul(x, W)
    nbrs32 = nbrs.astype(jnp.int32)
    nbrs_pad = jnp.zeros((_NPAD, _DEG), jnp.int32).at[:_N].set(nbrs32)
    out = _sc_median(h, nbrs_pad.reshape(-1))
    return out[:_N]


def kernel(x, nbrs, W):
    return _run(x, nbrs, W)
